# initial kernel scaffold (unmeasured)
import jax
import jax.numpy as jnp
from jax import lax
from jax.experimental import pallas as pl
from jax.experimental.pallas import tpu as pltpu


def kernel(
    x,
):
    def body(*refs):
        pass

    out_shape = jax.ShapeDtypeStruct(..., jnp.float32)
    return pl.pallas_call(body, out_shape=out_shape)(...)


# baseline (device time: 109141 ns/iter reference)
import jax
import jax.numpy as jnp
from jax import lax
from jax.experimental import pallas as pl
from jax.experimental.pallas import tpu as pltpu

CHUNK = 1024


def kernel(x):
    m, n = x.shape
    half = n // 2
    out_m = 2 * m
    n_chunks = m // CHUNK

    def body(x_hbm, out_ref, send_buf, cvt, cvt_sem, send_sem, recv_sem):
        my_x = lax.axis_index("x")
        my_y = lax.axis_index("y")
        my_z = lax.axis_index("z")
        partner = (1 - my_x, my_y, my_z)

        barrier_sem = pltpu.get_barrier_semaphore()
        pl.semaphore_signal(
            barrier_sem, inc=1, device_id=partner,
            device_id_type=pl.DeviceIdType.MESH,
        )
        pl.semaphore_wait(barrier_sem, 1)

        def run(my_col, partner_col, my_row0, partner_row0):
            for c in range(n_chunks):
                cp = pltpu.make_async_copy(
                    x_hbm.at[pl.ds(c * CHUNK, CHUNK), pl.ds(partner_col, half)],
                    cvt,
                    cvt_sem,
                )
                cp.start()
                cp.wait()
                send_buf[pl.ds(c * CHUNK, CHUNK), :] = (
                    cvt[...].astype(jnp.bfloat16)
                )

            rdma = pltpu.make_async_remote_copy(
                src_ref=send_buf,
                dst_ref=out_ref.at[pl.ds(my_row0, m), :],
                send_sem=send_sem,
                recv_sem=recv_sem,
                device_id=partner,
                device_id_type=pl.DeviceIdType.MESH,
            )
            rdma.start()

            for c in range(n_chunks):
                cp = pltpu.make_async_copy(
                    x_hbm.at[pl.ds(c * CHUNK, CHUNK), pl.ds(my_col, half)],
                    cvt,
                    cvt_sem,
                )
                cp.start()
                cp.wait()
                out_ref[pl.ds(my_row0 + c * CHUNK, CHUNK), :] = (
                    cvt[...].astype(jnp.bfloat16)
                )

            rdma.wait()

        @pl.when(my_x == 0)
        def _():
            run(my_col=0, partner_col=half, my_row0=0, partner_row0=m)

        @pl.when(my_x == 1)
        def _():
            run(my_col=half, partner_col=0, my_row0=m, partner_row0=0)

    return pl.pallas_call(
        body,
        out_shape=jax.ShapeDtypeStruct((out_m, half), jnp.bfloat16),
        in_specs=[pl.BlockSpec(memory_space=pl.ANY)],
        out_specs=pl.BlockSpec(memory_space=pltpu.VMEM),
        scratch_shapes=[
            pltpu.VMEM((m, half), jnp.bfloat16),
            pltpu.VMEM((CHUNK, half), jnp.float32),
            pltpu.SemaphoreType.DMA,
            pltpu.SemaphoreType.DMA,
            pltpu.SemaphoreType.DMA,
        ],
        compiler_params=pltpu.CompilerParams(collective_id=0),
    )(x)


# device time: 101105 ns/iter; 1.0795x vs baseline; 1.0795x over previous
import jax
import jax.numpy as jnp
from jax import lax
from jax.experimental import pallas as pl
from jax.experimental.pallas import tpu as pltpu

CHUNK = 512


def kernel(x):
    m, n = x.shape
    half = n // 2
    out_m = 2 * m
    n_chunks = m // CHUNK

    def body(x_hbm, out_ref, send_buf, cvt,
             cvt_sems, send_sems, recv_sems):
        my_x = lax.axis_index("x")
        my_y = lax.axis_index("y")
        my_z = lax.axis_index("z")
        partner = (1 - my_x, my_y, my_z)

        barrier_sem = pltpu.get_barrier_semaphore()
        pl.semaphore_signal(
            barrier_sem, inc=1, device_id=partner,
            device_id_type=pl.DeviceIdType.MESH,
        )
        pl.semaphore_wait(barrier_sem, 1)

        def stage_in(c, col):
            return pltpu.make_async_copy(
                x_hbm.at[pl.ds(c * CHUNK, CHUNK), pl.ds(col, half)],
                cvt.at[c % 2],
                cvt_sems.at[c % 2],
            )

        def run(my_col, partner_col, my_row0, partner_row0):
            rdmas = []
            stage_in(0, partner_col).start()
            for c in range(n_chunks):
                if c + 1 < n_chunks:
                    stage_in(c + 1, partner_col).start()
                stage_in(c, partner_col).wait()
                send_buf[pl.ds(c * CHUNK, CHUNK), :] = (
                    cvt[c % 2].astype(jnp.bfloat16)
                )
                rdma = pltpu.make_async_remote_copy(
                    src_ref=send_buf.at[pl.ds(c * CHUNK, CHUNK), :],
                    dst_ref=out_ref.at[pl.ds(my_row0 + c * CHUNK, CHUNK), :],
                    send_sem=send_sems.at[c],
                    recv_sem=recv_sems.at[c],
                    device_id=partner,
                    device_id_type=pl.DeviceIdType.MESH,
                )
                rdma.start()
                rdmas.append(rdma)

            stage_in(0, my_col).start()
            for c in range(n_chunks):
                if c + 1 < n_chunks:
                    stage_in(c + 1, my_col).start()
                stage_in(c, my_col).wait()
                out_ref[pl.ds(my_row0 + c * CHUNK, CHUNK), :] = (
                    cvt[c % 2].astype(jnp.bfloat16)
                )

            for rdma in rdmas:
                rdma.wait()

        @pl.when(my_x == 0)
        def _():
            run(my_col=0, partner_col=half, my_row0=0, partner_row0=m)

        @pl.when(my_x == 1)
        def _():
            run(my_col=half, partner_col=0, my_row0=m, partner_row0=0)

    return pl.pallas_call(
        body,
        out_shape=jax.ShapeDtypeStruct((out_m, half), jnp.bfloat16),
        in_specs=[pl.BlockSpec(memory_space=pl.ANY)],
        out_specs=pl.BlockSpec(memory_space=pltpu.VMEM),
        scratch_shapes=[
            pltpu.VMEM((m, half), jnp.bfloat16),
            pltpu.VMEM((2, CHUNK, half), jnp.float32),
            pltpu.SemaphoreType.DMA((2,)),
            pltpu.SemaphoreType.DMA((n_chunks,)),
            pltpu.SemaphoreType.DMA((n_chunks,)),
        ],
        compiler_params=pltpu.CompilerParams(collective_id=0),
    )(x)
